# baseline (speedup 1.0000x reference)
import jax
import jax.numpy as jnp
from jax.experimental import pallas as pl
from jax.experimental.pallas import tpu as pltpu


def _rsh(v):
    return jnp.concatenate([jnp.zeros_like(v[:, :1]), v[:, :-1]], axis=1)


def _csh(v):
    return jnp.concatenate([jnp.zeros_like(v[:, :, :1]), v[:, :, :-1]], axis=2)


def _taps_from_phases(p00, p01, p10, p11):
    return [
        _csh(_rsh(p11)), _rsh(p10), _rsh(p11),
        _csh(p01),       p00,       p01,
        _csh(p11),       p10,       p11,
    ]


def _taps_s1(a):
    zr = jnp.zeros_like(a[:, :1])
    rows = (jnp.concatenate([zr, a[:, :-1]], axis=1),
            a,
            jnp.concatenate([a[:, 1:], zr], axis=1))
    taps = []
    for r in rows:
        zc = jnp.zeros_like(r[:, :, :1])
        taps += [jnp.concatenate([zc, r[:, :, :-1]], axis=2), r,
                 jnp.concatenate([r[:, :, 1:], zc], axis=2)]
    return taps


def _conv(taps, w_ref, shift_ref, *, relu):
    b, h, w, c = taps[0].shape
    patches = jnp.concatenate(taps, axis=-1).reshape(b * h * w, 9 * c)
    y = jnp.dot(patches.astype(jnp.bfloat16), w_ref[...],
                preferred_element_type=jnp.float32)
    y = y + shift_ref[...]
    return jnp.maximum(y, 0.0) if relu else y


def _block(phases, w1, w2, wsc, s1, s2, ssc):
    p00 = phases[0]
    b, oh, ow, c = p00.shape
    cout = w2.shape[-1]

    hid = _conv(_taps_from_phases(*phases), w1, s1, relu=True)
    sc = jnp.dot(p00.reshape(b * oh * ow, c).astype(jnp.bfloat16), wsc[...],
                 preferred_element_type=jnp.float32) + ssc[...]
    h4 = hid.reshape(b, oh, ow, cout)
    y = _conv(_taps_s1(h4), w2, s2, relu=False)
    return jnp.maximum(y + sc, 0.0)


def _read_phases(ref, oh, ow):
    return [ref[:, pl.ds(ph, oh, 2), pl.ds(pw, ow, 2), :]
            for ph in (0, 1) for pw in (0, 1)]


def _route_kernel(x_ref, *refs):
    prm = refs[:18]
    out_ref = refs[18]
    scr1, scr2 = refs[19], refs[20]

    b, h = x_ref.shape[0], x_ref.shape[1]
    act1 = _block(_read_phases(x_ref, h // 2, h // 2), *prm[0:6])
    scr1[...] = act1.reshape(b, 16, 16, 64)
    act2 = _block(_read_phases(scr1, 8, 8), *prm[6:12])
    scr2[...] = act2.reshape(b, 8, 8, 128)
    act3 = _block(_read_phases(scr2, 4, 4), *prm[12:18])

    out_ref[...] = jnp.mean(act3.reshape(b, 16, 256), axis=1)


def _zero_map(ndim):
    return lambda i: (0,) * ndim


def kernel(x, b1_sel1, b1_sel2, b1_w1, b1_w2, b1_wsc, b1_bias1, b1_bias2,
           b1_bsc, b2_sel1, b2_sel2, b2_w1, b2_w2, b2_wsc, b2_bias1,
           b2_bias2, b2_bsc, b3_sel1, b3_sel2, b3_w1, b3_w2, b3_wsc,
           b3_bias1, b3_bias2, b3_bsc):
    n, c, h, w = x.shape
    xh = jnp.transpose(x, (0, 2, 3, 1))
    flat = (b1_w1, b1_w2, b1_wsc, b1_bias1, b1_bias2, b1_bsc,
            b2_w1, b2_w2, b2_wsc, b2_bias1, b2_bias2, b2_bsc,
            b3_w1, b3_w2, b3_wsc, b3_bias1, b3_bias2, b3_bsc)
    cout = b3_bias2.shape[-1]

    bk = next(d for d in (8, 4, 2, 1) if n % d == 0)
    in_specs = [pl.BlockSpec((bk, h, w, c), lambda i: (i, 0, 0, 0))]
    in_specs += [pl.BlockSpec(arr.shape, _zero_map(arr.ndim)) for arr in flat]

    pooled = pl.pallas_call(
        _route_kernel,
        out_shape=jax.ShapeDtypeStruct((n, cout), jnp.float32),
        grid=(n // bk,),
        in_specs=in_specs,
        out_specs=pl.BlockSpec((bk, cout), lambda i: (i, 0)),
        scratch_shapes=[pltpu.VMEM((bk, 16, 16, 64), jnp.float32),
                        pltpu.VMEM((bk, 8, 8, 128), jnp.float32)],
        compiler_params=pltpu.CompilerParams(dimension_semantics=("parallel",)),
    )(xh, *flat)
    return pooled.reshape(n, cout, 1, 1)


# speedup vs baseline: 1.0832x; 1.0832x over previous
import jax
import jax.numpy as jnp
from jax.experimental import pallas as pl
from jax.experimental.pallas import tpu as pltpu


def _rsh(v):
    return jnp.concatenate([jnp.zeros_like(v[:, :1]), v[:, :-1]], axis=1)


def _csh(v):
    return jnp.concatenate([jnp.zeros_like(v[:, :, :1]), v[:, :, :-1]], axis=2)


def _taps_from_phases(p00, p01, p10, p11):
    return [
        _csh(_rsh(p11)), _rsh(p10), _rsh(p11),
        _csh(p01),       p00,       p01,
        _csh(p11),       p10,       p11,
    ]


def _taps_s1(a):
    zr = jnp.zeros_like(a[:, :1])
    rows = (jnp.concatenate([zr, a[:, :-1]], axis=1),
            a,
            jnp.concatenate([a[:, 1:], zr], axis=1))
    taps = []
    for r in rows:
        zc = jnp.zeros_like(r[:, :, :1])
        taps += [jnp.concatenate([zc, r[:, :, :-1]], axis=2), r,
                 jnp.concatenate([r[:, :, 1:], zc], axis=2)]
    return taps


def _conv(taps, w_ref, shift_ref, *, relu):
    b, h, w, c = taps[0].shape
    patches = jnp.concatenate(taps, axis=-1).reshape(b * h * w, 9 * c)
    y = jnp.dot(patches.astype(jnp.bfloat16), w_ref[...],
                preferred_element_type=jnp.float32)
    y = y + shift_ref[...]
    return jnp.maximum(y, 0.0) if relu else y


def _block(phases, w1, w2, wsc, s1, s2, ssc):
    p00 = phases[0]
    b, oh, ow, c = p00.shape
    cout = w2.shape[-1]

    hid = _conv(_taps_from_phases(*phases), w1, s1, relu=True)
    sc = jnp.dot(p00.reshape(b * oh * ow, c).astype(jnp.bfloat16), wsc[...],
                 preferred_element_type=jnp.float32) + ssc[...]
    h4 = hid.reshape(b, oh, ow, cout)
    y = _conv(_taps_s1(h4), w2, s2, relu=False)
    return jnp.maximum(y + sc, 0.0)


def _read_phases(ref, oh, ow):
    return [ref[:, pl.ds(ph, oh, 2), pl.ds(pw, ow, 2), :]
            for ph in (0, 1) for pw in (0, 1)]


def _route_kernel(x_ref, *refs):
    prm = refs[:18]
    out_ref = refs[18]
    scr1, scr2 = refs[19], refs[20]

    b, h = x_ref.shape[0], x_ref.shape[1]
    act1 = _block(_read_phases(x_ref, h // 2, h // 2), *prm[0:6])
    scr1[...] = act1.reshape(b, 16, 16, 64)
    act2 = _block(_read_phases(scr1, 8, 8), *prm[6:12])
    scr2[...] = act2.reshape(b, 8, 8, 128)
    act3 = _block(_read_phases(scr2, 4, 4), *prm[12:18])

    out_ref[...] = jnp.mean(act3.reshape(b, 16, 256), axis=1)


def _zero_map(ndim):
    return lambda i: (0,) * ndim


def kernel(x, b1_sel1, b1_sel2, b1_w1, b1_w2, b1_wsc, b1_bias1, b1_bias2,
           b1_bsc, b2_sel1, b2_sel2, b2_w1, b2_w2, b2_wsc, b2_bias1,
           b2_bias2, b2_bsc, b3_sel1, b3_sel2, b3_w1, b3_w2, b3_wsc,
           b3_bias1, b3_bias2, b3_bsc):
    n, c, h, w = x.shape
    xh = jnp.transpose(x, (0, 2, 3, 1))
    flat = (b1_w1, b1_w2, b1_wsc, b1_bias1, b1_bias2, b1_bsc,
            b2_w1, b2_w2, b2_wsc, b2_bias1, b2_bias2, b2_bsc,
            b3_w1, b3_w2, b3_wsc, b3_bias1, b3_bias2, b3_bsc)
    cout = b3_bias2.shape[-1]

    bk = next(d for d in (32, 16, 8, 4, 2, 1) if n % d == 0)
    in_specs = [pl.BlockSpec((bk, h, w, c), lambda i: (i, 0, 0, 0))]
    in_specs += [pl.BlockSpec(arr.shape, _zero_map(arr.ndim)) for arr in flat]

    pooled = pl.pallas_call(
        _route_kernel,
        out_shape=jax.ShapeDtypeStruct((n, cout), jnp.float32),
        grid=(n // bk,),
        in_specs=in_specs,
        out_specs=pl.BlockSpec((bk, cout), lambda i: (i, 0)),
        scratch_shapes=[pltpu.VMEM((bk, 16, 16, 64), jnp.float32),
                        pltpu.VMEM((bk, 8, 8, 128), jnp.float32)],
        compiler_params=pltpu.CompilerParams(dimension_semantics=("parallel",)),
    )(xh, *flat)
    return pooled.reshape(n, cout, 1, 1)
